# baseline (speedup 1.0000x reference)
import functools

import jax
import jax.numpy as jnp
from jax import lax
from jax.experimental import pallas as pl
from jax.experimental.pallas import tpu as pltpu
from jax.experimental.pallas import tpu_sc as plsc

CONTENT_DIM = 64
TIME_DIM = 16
OUT_DIM = CONTENT_DIM + TIME_DIM

_INFO = plsc.get_sparse_core_info()
NC, NS = _INFO.num_cores, _INFO.num_subcores
NW = NC * NS

CHUNK = 128


def _tokenizer_body(n_per_w, ids_hbm, content_hbm, time_hbm, out_hbm,
                    idx_v, rows_c, rows_t, sem_c, sem_t):
    wid = lax.axis_index("s") * NC + lax.axis_index("c")
    base_w = wid * n_per_w
    n_chunks = n_per_w // CHUNK

    def chunk_body(i, carry):
        base = base_w + i * CHUNK
        pltpu.sync_copy(ids_hbm.at[pl.ds(base, CHUNK)], idx_v)
        cp_c = pltpu.async_copy(content_hbm.at[idx_v], rows_c, sem_c)
        cp_t = pltpu.async_copy(time_hbm.at[idx_v], rows_t, sem_t)
        cp_c.wait()
        cp_t.wait()
        pltpu.sync_copy(
            rows_c, out_hbm.at[pl.ds(base, CHUNK), pl.ds(0, CONTENT_DIM)])
        pltpu.sync_copy(
            rows_t, out_hbm.at[pl.ds(base, CHUNK), pl.ds(CONTENT_DIM, TIME_DIM)])
        return carry

    lax.fori_loop(0, n_chunks, chunk_body, 0)


@jax.jit
def kernel(token_ids, content_table, time_table):
    batch, seq = token_ids.shape
    n = batch * seq
    n_per_w = n // NW
    ids = token_ids.reshape(n).astype(jnp.int32)

    mesh = plsc.VectorSubcoreMesh(core_axis_name="c", subcore_axis_name="s")
    out = pl.kernel(
        functools.partial(_tokenizer_body, n_per_w),
        out_type=jax.ShapeDtypeStruct((n, OUT_DIM), jnp.float32),
        mesh=mesh,
        scratch_types=[
            pltpu.VMEM((CHUNK,), jnp.int32),
            pltpu.VMEM((CHUNK, CONTENT_DIM), jnp.float32),
            pltpu.VMEM((CHUNK, TIME_DIM), jnp.float32),
            pltpu.SemaphoreType.DMA,
            pltpu.SemaphoreType.DMA,
        ],
        compiler_params=pltpu.CompilerParams(use_tc_tiling_on_sc=False),
    )(ids, content_table, time_table)
    return out.reshape(batch, seq, OUT_DIM)
